# baseline (speedup 1.0000x reference)
import jax
import jax.numpy as jnp
from jax.experimental import pallas as pl
from jax.experimental.pallas import tpu as pltpu

_INV_TEMP = 0.5
_BG = 1024
_Q = 4
_BT = _BG // _Q
_NBUF = 4


def _start_group(x_hbm, bufs, in_sems, group, slot):
    for q in range(_Q):
        pltpu.make_async_copy(
            x_hbm.at[pl.ds((group * _Q + q) * _BT, _BT), :],
            bufs[slot].at[pl.ds(q * _BT, _BT), :],
            in_sems.at[slot, q],
        ).start()


def _router_body(x_hbm, w_ref, b_ref, o_hbm, *scratch):
    bufs = scratch[:_NBUF]
    outs = scratch[_NBUF:2 * _NBUF]
    in_sems = scratch[2 * _NBUF]
    out_sems = scratch[2 * _NBUF + 1]
    i = pl.program_id(0)
    n = pl.num_programs(0)

    @pl.when(i == 0)
    def _prologue():
        for s in range(_NBUF):
            _start_group(x_hbm, bufs, in_sems, s, s)

    w = w_ref[...].astype(jnp.bfloat16)
    for j in range(_NBUF):
        group = i * _NBUF + j
        for q in range(_Q):
            pltpu.make_async_copy(
                x_hbm.at[pl.ds((group * _Q + q) * _BT, _BT), :],
                bufs[j].at[pl.ds(q * _BT, _BT), :],
                in_sems.at[j, q],
            ).wait()

        prev = (j + _NBUF - 1) % _NBUF
        nxt = group + _NBUF - 1

        @pl.when(jnp.logical_and(group >= 1, nxt < n * _NBUF))
        def _refill(nxt=nxt, prev=prev):
            _start_group(x_hbm, bufs, in_sems, nxt, prev)

        logits = jax.lax.dot_general(
            bufs[j][...].astype(jnp.bfloat16), w,
            dimension_numbers=(((1,), (1,)), ((), ())),
            preferred_element_type=jnp.float32,
        )
        logits = (logits + b_ref[...]) * _INV_TEMP
        m = jnp.max(logits, axis=-1, keepdims=True)
        e = jnp.exp(logits - m)

        @pl.when(i > 0)
        def _drain(j=j):
            pltpu.make_async_copy(
                outs[j], o_hbm.at[pl.ds(0, _BG), :], out_sems.at[j]
            ).wait()

        outs[j][...] = e * (1.0 / jnp.sum(e, axis=-1, keepdims=True))
        pltpu.make_async_copy(
            outs[j], o_hbm.at[pl.ds(group * _BG, _BG), :], out_sems.at[j]
        ).start(priority=1)

    @pl.when(i == n - 1)
    def _epilogue():
        for s in range(_NBUF):
            pltpu.make_async_copy(
                outs[s], o_hbm.at[pl.ds(0, _BG), :], out_sems.at[s]
            ).wait()


def kernel(x, gate_w, gate_b):
    n_tokens, d = x.shape
    ne = gate_w.shape[0]
    b2d = gate_b.reshape(1, ne)
    return pl.pallas_call(
        _router_body,
        grid=(n_tokens // (_NBUF * _BG),),
        in_specs=[
            pl.BlockSpec(memory_space=pltpu.MemorySpace.HBM),
            pl.BlockSpec((ne, d), lambda i: (0, 0)),
            pl.BlockSpec((1, ne), lambda i: (0, 0)),
        ],
        out_specs=pl.BlockSpec(memory_space=pltpu.MemorySpace.HBM),
        out_shape=jax.ShapeDtypeStruct((n_tokens, ne), jnp.float32),
        scratch_shapes=(
            [pltpu.VMEM((_BG, d), jnp.float32)] * _NBUF
            + [pltpu.VMEM((_BG, ne), jnp.float32)] * _NBUF
            + [pltpu.SemaphoreType.DMA((_NBUF, _Q)),
               pltpu.SemaphoreType.DMA((_NBUF,))]
        ),
    )(x, gate_w, b2d)


# speedup vs baseline: 1.0623x; 1.0623x over previous
import jax
import jax.numpy as jnp
from jax.experimental import pallas as pl

_INV_TEMP = 0.5
_BT = 1024


def _router_block(x_ref, w_ref, b_ref, o_ref):
    logits = jax.lax.dot_general(
        x_ref[...].astype(jnp.bfloat16), w_ref[...].astype(jnp.bfloat16),
        dimension_numbers=(((1,), (1,)), ((), ())),
        preferred_element_type=jnp.float32,
    )
    logits = (logits + b_ref[...]) * _INV_TEMP
    m = jnp.max(logits, axis=-1, keepdims=True)
    e = jnp.exp(logits - m)
    o_ref[...] = e * (1.0 / jnp.sum(e, axis=-1, keepdims=True))


def kernel(x, gate_w, gate_b):
    n_tokens, d = x.shape
    ne = gate_w.shape[0]
    b2d = gate_b.reshape(1, ne)
    return pl.pallas_call(
        _router_block,
        grid=(n_tokens // _BT,),
        in_specs=[
            pl.BlockSpec((_BT, d), lambda i: (i, 0)),
            pl.BlockSpec((ne, d), lambda i: (0, 0)),
            pl.BlockSpec((1, ne), lambda i: (0, 0)),
        ],
        out_specs=pl.BlockSpec((_BT, ne), lambda i: (i, 0)),
        out_shape=jax.ShapeDtypeStruct((n_tokens, ne), jnp.float32),
    )(x, gate_w, b2d)
